# baseline (speedup 1.0000x reference)
import functools

import jax
import jax.numpy as jnp
from jax import lax
from jax.experimental import pallas as pl
from jax.experimental.pallas import tpu as pltpu
from jax.experimental.pallas import tpu_sc as plsc

_B, _N, _IN, _HID, _OUT, _K = 16, 1024, 768, 64, 64, 1024
_ROWS = _B * _N
_BLK = 1024
_GRID = _ROWS // _BLK

_NC, _NS = 2, 16
_NW = _NC * _NS
_RPW = _ROWS // _NW
_CH = 128
_NCH = _RPW // _CH


def _tc_body(obs_ref, w1_ref, b1_ref, w2_ref, b2_ref, w3_ref, b3_ref,
             cb_ref, idx_ref, loss_ref):
    h = jnp.maximum(
        jnp.dot(obs_ref[...], w1_ref[...], preferred_element_type=jnp.float32)
        + b1_ref[...], 0.0)
    h = jnp.maximum(
        jnp.dot(h, w2_ref[...], preferred_element_type=jnp.float32)
        + b2_ref[...], 0.0)
    z = (jnp.dot(h, w3_ref[...], preferred_element_type=jnp.float32)
         + b3_ref[...])
    cb = cb_ref[...]
    zn = jnp.sum(z * z, axis=1, keepdims=True)
    cn = jnp.sum(cb * cb, axis=1)
    zc = lax.dot_general(z, cb, (((1,), (1,)), ((), ())),
                         preferred_element_type=jnp.float32)
    d = zn - 2.0 * zc + cn[None, :]
    dmin = jnp.min(d, axis=1)
    ids = lax.broadcasted_iota(jnp.int32, d.shape, 1)
    idx = jnp.min(jnp.where(d == dmin[:, None], ids, _K), axis=1)
    idx_ref[0, 0, :] = idx

    @pl.when(pl.program_id(0) == 0)
    def _():
        loss_ref[0, 0] = 0.0

    loss_ref[0, 0] += jnp.sum(dmin)


def _tc_main(obs_flat, w1, b1, w2, b2, w3, b3, codebook):
    full = lambda s: pl.BlockSpec(s, lambda i: (0,) * len(s))
    return pl.pallas_call(
        _tc_body,
        grid=(_GRID,),
        in_specs=[
            pl.BlockSpec((_BLK, _IN), lambda i: (i, 0)),
            full((_IN, _HID)), full((1, _HID)),
            full((_HID, _HID)), full((1, _HID)),
            full((_HID, _OUT)), full((1, _OUT)),
            full((_K, _OUT)),
        ],
        out_specs=[
            pl.BlockSpec((1, 1, _BLK), lambda i: (i, 0, 0)),
            pl.BlockSpec(memory_space=pltpu.SMEM),
        ],
        out_shape=[
            jax.ShapeDtypeStruct((_GRID, 1, _BLK), jnp.int32),
            jax.ShapeDtypeStruct((1, 1), jnp.float32),
        ],
    )(obs_flat, w1, b1, w2, b2, w3, b3, codebook)


@functools.cache
def _make_sc_gather():
    @functools.partial(
        pl.kernel,
        mesh=plsc.VectorSubcoreMesh(core_axis_name="c", subcore_axis_name="s"),
        compiler_params=pltpu.CompilerParams(use_tc_tiling_on_sc=False),
        out_type=jax.ShapeDtypeStruct((_ROWS, _OUT), jnp.float32),
    scratch_types=[
            pltpu.VMEM((_RPW,), jnp.int32),
            pltpu.VMEM((_RPW, _OUT), jnp.float32),
            pltpu.SemaphoreType.DMA,
        ],
    )
    def _sc_gather(cb_hbm, idx_hbm, out_hbm, idx_v, rows_v, sem):
        wid = lax.axis_index("s") * _NC + lax.axis_index("c")
        base = wid * _RPW
        pltpu.sync_copy(idx_hbm.at[pl.ds(base, _RPW)], idx_v)
        pltpu.async_copy(cb_hbm.at[idx_v], rows_v, sem).wait()
        pltpu.sync_copy(rows_v, out_hbm.at[pl.ds(base, _RPW)])

    return _sc_gather


def kernel(obs, W1, b1, W2, b2, W3, b3, codebook):
    obs_flat = obs.reshape(_ROWS, _IN)
    idx3, loss_sum = _tc_main(
        obs_flat, W1, b1.reshape(1, _HID), W2, b2.reshape(1, _HID),
        W3, b3.reshape(1, _OUT), codebook)
    idx_flat = idx3.reshape(_ROWS)
    msg_flat = _make_sc_gather()(codebook, idx_flat)
    msg = msg_flat.reshape(_B, _N, _OUT)
    idx_out = idx3.reshape(_B, _N)
    cmt_loss = loss_sum[0, 0] / jnp.float32(_ROWS * _OUT)
    return (msg, idx_out, cmt_loss)


# speedup vs baseline: 1.0917x; 1.0917x over previous
import functools

import jax
import jax.numpy as jnp
from jax import lax
from jax.experimental import pallas as pl
from jax.experimental.pallas import tpu as pltpu
from jax.experimental.pallas import tpu_sc as plsc

_B, _N, _IN, _HID, _OUT, _K = 16, 1024, 768, 64, 64, 1024
_ROWS = _B * _N
_BLK = 1024
_GRID = _ROWS // _BLK

_NC, _NS = 2, 16
_NW = _NC * _NS
_RPW = _ROWS // _NW
_CH = 128
_NCH = _RPW // _CH


_KC = 256
_NKC = _K // _KC


def _tc_body(obs_ref, w1_ref, b1_ref, w2_ref, b2_ref, w3_ref, b3_ref,
             cb_ref, idx_ref, loss_ref):
    h = jnp.maximum(
        jnp.dot(obs_ref[...], w1_ref[...], preferred_element_type=jnp.float32)
        + b1_ref[...], 0.0)
    h = jnp.maximum(
        jnp.dot(h, w2_ref[...], preferred_element_type=jnp.float32)
        + b2_ref[...], 0.0)
    z = (jnp.dot(h, w3_ref[...], preferred_element_type=jnp.float32)
         + b3_ref[...])
    zm2 = -2.0 * z
    zn = jnp.sum(z * z, axis=1, keepdims=True)
    cb = cb_ref[...]
    cn = jnp.sum(cb * cb, axis=1)
    s = lax.dot_general(zm2, cb, (((1,), (1,)), ((), ())),
                        preferred_element_type=jnp.float32)
    d = (zn + s) + cn[None, :]
    idx = jnp.argmin(d, axis=1)
    dmin = jnp.min(d, axis=1)
    idx_ref[0, 0, :] = idx

    @pl.when(pl.program_id(0) == 0)
    def _():
        loss_ref[0, 0] = 0.0

    loss_ref[0, 0] += jnp.sum(dmin)


def _tc_main(obs_flat, w1, b1, w2, b2, w3, b3, codebook):
    full = lambda s: pl.BlockSpec(s, lambda i: (0,) * len(s))
    return pl.pallas_call(
        _tc_body,
        grid=(_GRID,),
        in_specs=[
            pl.BlockSpec((_BLK, _IN), lambda i: (i, 0)),
            full((_IN, _HID)), full((1, _HID)),
            full((_HID, _HID)), full((1, _HID)),
            full((_HID, _OUT)), full((1, _OUT)),
            full((_K, _OUT)),
        ],
        out_specs=[
            pl.BlockSpec((1, 1, _BLK), lambda i: (i, 0, 0)),
            pl.BlockSpec(memory_space=pltpu.SMEM),
        ],
        out_shape=[
            jax.ShapeDtypeStruct((_GRID, 1, _BLK), jnp.int32),
            jax.ShapeDtypeStruct((1, 1), jnp.float32),
        ],
    )(obs_flat, w1, b1, w2, b2, w3, b3, codebook)


@functools.cache
def _make_sc_gather():
    @functools.partial(
        pl.kernel,
        mesh=plsc.VectorSubcoreMesh(core_axis_name="c", subcore_axis_name="s"),
        compiler_params=pltpu.CompilerParams(use_tc_tiling_on_sc=False,
                                             needs_layout_passes=False),
        out_type=jax.ShapeDtypeStruct((_ROWS, _OUT), jnp.float32),
    scratch_types=[
            pltpu.VMEM((_K, _OUT), jnp.float32),
            pltpu.VMEM((_RPW,), jnp.int32),
            pltpu.VMEM((_RPW, _OUT), jnp.float32),
            pltpu.SemaphoreType.DMA,
        ],
    )
    def _sc_gather(cb_hbm, idx_hbm, out_hbm, cb_v, idx_v, rows_v, sem):
        wid = lax.axis_index("s") * _NC + lax.axis_index("c")
        base = wid * _RPW
        cb_cp = pltpu.async_copy(cb_hbm, cb_v, sem)
        pltpu.sync_copy(idx_hbm.at[pl.ds(base, _RPW)], idx_v)
        cb_cp.wait()
        lanes = lax.iota(jnp.int32, 16)

        def body(g, carry):
            idxv = idx_v[pl.ds(g * 16, 16)]
            rows = g * 16 + lanes
            for j in range(_OUT):
                col = jnp.full((16,), j, jnp.int32)
                vals = plsc.load_gather(cb_v, [idxv, col])
                plsc.store_scatter(rows_v, [rows, col], vals)
            return carry

        lax.fori_loop(0, _RPW // 16, body, 0)
        pltpu.sync_copy(rows_v, out_hbm.at[pl.ds(base, _RPW)])

    return _sc_gather


def kernel(obs, W1, b1, W2, b2, W3, b3, codebook):
    obs_flat = obs.reshape(_ROWS, _IN)
    idx3, loss_sum = _tc_main(
        obs_flat, W1, b1.reshape(1, _HID), W2, b2.reshape(1, _HID),
        W3, b3.reshape(1, _OUT), codebook)
    idx_flat = idx3.reshape(_ROWS)
    msg_flat = _make_sc_gather()(codebook, idx_flat)
    msg = msg_flat.reshape(_B, _N, _OUT)
    idx_out = idx3.reshape(_B, _N)
    cmt_loss = loss_sum[0, 0] / jnp.float32(_ROWS * _OUT)
    return (msg, idx_out, cmt_loss)


# speedup vs baseline: 1.1574x; 1.0601x over previous
import functools

import jax
import jax.numpy as jnp
from jax import lax
from jax.experimental import pallas as pl
from jax.experimental.pallas import tpu as pltpu
from jax.experimental.pallas import tpu_sc as plsc

_B, _N, _IN, _HID, _OUT, _K = 16, 1024, 768, 64, 64, 1024
_ROWS = _B * _N
_BLK = 1024
_GRID = _ROWS // _BLK

_NC, _NS = 2, 16
_NW = _NC * _NS
_RPW = _ROWS // _NW
_CH = 128
_NCH = _RPW // _CH


_KC = 256
_NKC = _K // _KC


def _tc_body(obs_ref, w1_ref, b1_ref, w2_ref, b2_ref, w3_ref, b3_ref,
             cb_ref, idx_ref, loss_ref):
    h = jnp.maximum(
        jnp.dot(obs_ref[...], w1_ref[...], preferred_element_type=jnp.float32)
        + b1_ref[...], 0.0)
    h = jnp.maximum(
        jnp.dot(h, w2_ref[...], preferred_element_type=jnp.float32)
        + b2_ref[...], 0.0)
    z = (jnp.dot(h, w3_ref[...], preferred_element_type=jnp.float32)
         + b3_ref[...])
    zm2 = -2.0 * z
    zn = jnp.sum(z * z, axis=1, keepdims=True)
    cb = cb_ref[...]
    cn = jnp.sum(cb * cb, axis=1)
    s = lax.dot_general(zm2, cb, (((1,), (1,)), ((), ())),
                        preferred_element_type=jnp.float32)
    d = (zn + s) + cn[None, :]
    idx = jnp.argmin(d, axis=1)
    dmin = jnp.min(d, axis=1)
    idx_ref[0, 0, :] = idx

    @pl.when(pl.program_id(0) == 0)
    def _():
        loss_ref[0, 0] = 0.0

    loss_ref[0, 0] += jnp.sum(dmin)


def _tc_main(obs_flat, w1, b1, w2, b2, w3, b3, codebook):
    full = lambda s: pl.BlockSpec(s, lambda i: (0,) * len(s))
    return pl.pallas_call(
        _tc_body,
        grid=(_GRID,),
        in_specs=[
            pl.BlockSpec((_BLK, _IN), lambda i: (i, 0)),
            full((_IN, _HID)), full((1, _HID)),
            full((_HID, _HID)), full((1, _HID)),
            full((_HID, _OUT)), full((1, _OUT)),
            full((_K, _OUT)),
        ],
        out_specs=[
            pl.BlockSpec((1, 1, _BLK), lambda i: (i, 0, 0)),
            pl.BlockSpec(memory_space=pltpu.SMEM),
        ],
        out_shape=[
            jax.ShapeDtypeStruct((_GRID, 1, _BLK), jnp.int32),
            jax.ShapeDtypeStruct((1, 1), jnp.float32),
        ],
    )(obs_flat, w1, b1, w2, b2, w3, b3, codebook)


@functools.cache
def _make_sc_gather():
    @functools.partial(
        pl.kernel,
        mesh=plsc.VectorSubcoreMesh(core_axis_name="c", subcore_axis_name="s"),
        compiler_params=pltpu.CompilerParams(use_tc_tiling_on_sc=False,
                                             needs_layout_passes=False),
        out_type=jax.ShapeDtypeStruct((_ROWS, _OUT), jnp.float32),
    scratch_types=[
            pltpu.VMEM((_K, _OUT), jnp.float32),
            pltpu.VMEM((_RPW,), jnp.int32),
            pltpu.VMEM((_RPW, _OUT), jnp.float32),
            pltpu.SemaphoreType.DMA,
        ],
    )
    def _sc_gather(cb_hbm, idx_hbm, out_hbm, cb_v, idx_v, rows_v, sem):
        wid = lax.axis_index("s") * _NC + lax.axis_index("c")
        base = wid * _RPW
        cb_cp = pltpu.async_copy(cb_hbm, cb_v, sem)
        pltpu.sync_copy(idx_hbm.at[pl.ds(base, _RPW)], idx_v)
        cb_cp.wait()
        lanes = lax.iota(jnp.int32, 16)

        @plsc.parallel_loop(0, _RPW // 16, unroll=2)
        def body(g):
            idxv = idx_v[pl.ds(g * 16, 16)]
            rows = g * 16 + lanes
            for j0 in range(0, _OUT, 8):
                cols = [jnp.full((16,), j0 + t, jnp.int32) for t in range(8)]
                vals = [plsc.load_gather(cb_v, [idxv, c]) for c in cols]
                for c, v in zip(cols, vals):
                    plsc.store_scatter(rows_v, [rows, c], v)
        pltpu.sync_copy(rows_v, out_hbm.at[pl.ds(base, _RPW)])

    return _sc_gather


def kernel(obs, W1, b1, W2, b2, W3, b3, codebook):
    obs_flat = obs.reshape(_ROWS, _IN)
    idx3, loss_sum = _tc_main(
        obs_flat, W1, b1.reshape(1, _HID), W2, b2.reshape(1, _HID),
        W3, b3.reshape(1, _OUT), codebook)
    idx_flat = idx3.reshape(_ROWS)
    msg_flat = _make_sc_gather()(codebook, idx_flat)
    msg = msg_flat.reshape(_B, _N, _OUT)
    idx_out = idx3.reshape(_B, _N)
    cmt_loss = loss_sum[0, 0] / jnp.float32(_ROWS * _OUT)
    return (msg, idx_out, cmt_loss)


# speedup vs baseline: 1.1578x; 1.0004x over previous
import functools

import jax
import jax.numpy as jnp
from jax import lax
from jax.experimental import pallas as pl
from jax.experimental.pallas import tpu as pltpu
from jax.experimental.pallas import tpu_sc as plsc

_B, _N, _IN, _HID, _OUT, _K = 16, 1024, 768, 64, 64, 1024
_ROWS = _B * _N
_BLK = 1024
_GRID = _ROWS // _BLK

_NC, _NS = 2, 16
_NW = _NC * _NS
_RPW = _ROWS // _NW
_CH = 128
_NCH = _RPW // _CH


_KC = 256
_NKC = _K // _KC


def _tc_body(obs_ref, w1_ref, b1_ref, w2_ref, b2_ref, w3_ref, b3_ref,
             cb_ref, idx_ref, loss_ref):
    h = jnp.maximum(
        jnp.dot(obs_ref[...], w1_ref[...], preferred_element_type=jnp.float32)
        + b1_ref[...], 0.0)
    h = jnp.maximum(
        jnp.dot(h, w2_ref[...], preferred_element_type=jnp.float32)
        + b2_ref[...], 0.0)
    z = (jnp.dot(h, w3_ref[...], preferred_element_type=jnp.float32)
         + b3_ref[...])
    zm2 = -2.0 * z
    zn = jnp.sum(z * z, axis=1, keepdims=True)
    cb = cb_ref[...]
    cn = jnp.sum(cb * cb, axis=1)
    s = lax.dot_general(zm2, cb, (((1,), (1,)), ((), ())),
                        preferred_element_type=jnp.float32)
    d = (zn + s) + cn[None, :]
    idx = jnp.argmin(d, axis=1)
    dmin = jnp.min(d, axis=1)
    idx_ref[0, 0, :] = idx

    @pl.when(pl.program_id(0) == 0)
    def _():
        loss_ref[0, 0] = 0.0

    loss_ref[0, 0] += jnp.sum(dmin)


def _tc_main(obs_flat, w1, b1, w2, b2, w3, b3, codebook):
    full = lambda s: pl.BlockSpec(s, lambda i: (0,) * len(s))
    return pl.pallas_call(
        _tc_body,
        grid=(_GRID,),
        in_specs=[
            pl.BlockSpec((_BLK, _IN), lambda i: (i, 0)),
            full((_IN, _HID)), full((1, _HID)),
            full((_HID, _HID)), full((1, _HID)),
            full((_HID, _OUT)), full((1, _OUT)),
            full((_K, _OUT)),
        ],
        out_specs=[
            pl.BlockSpec((1, 1, _BLK), lambda i: (i, 0, 0)),
            pl.BlockSpec(memory_space=pltpu.SMEM),
        ],
        out_shape=[
            jax.ShapeDtypeStruct((_GRID, 1, _BLK), jnp.int32),
            jax.ShapeDtypeStruct((1, 1), jnp.float32),
        ],
    )(obs_flat, w1, b1, w2, b2, w3, b3, codebook)


@functools.cache
def _make_sc_gather():
    @functools.partial(
        pl.kernel,
        mesh=plsc.VectorSubcoreMesh(core_axis_name="c", subcore_axis_name="s"),
        compiler_params=pltpu.CompilerParams(use_tc_tiling_on_sc=False,
                                             needs_layout_passes=False),
        out_type=jax.ShapeDtypeStruct((_B, _N, _OUT), jnp.float32),
    scratch_types=[
            pltpu.VMEM((_K, _OUT), jnp.float32),
            pltpu.VMEM((_RPW,), jnp.int32),
            pltpu.VMEM((_RPW, _OUT), jnp.float32),
            pltpu.SemaphoreType.DMA,
        ],
    )
    def _sc_gather(cb_hbm, idx_hbm, out_hbm, cb_v, idx_v, rows_v, sem):
        wid = lax.axis_index("s") * _NC + lax.axis_index("c")
        base = wid * _RPW
        cb_cp = pltpu.async_copy(cb_hbm, cb_v, sem)
        pltpu.sync_copy(idx_hbm.at[pl.ds(base, _RPW)], idx_v)
        cb_cp.wait()
        lanes = lax.iota(jnp.int32, 16)

        @plsc.parallel_loop(0, _RPW // 16, unroll=2)
        def body(g):
            idxv = idx_v[pl.ds(g * 16, 16)]
            rows = g * 16 + lanes
            for j0 in range(0, _OUT, 16):
                cols = [jnp.full((16,), j0 + t, jnp.int32) for t in range(16)]
                vals = [plsc.load_gather(cb_v, [idxv, c]) for c in cols]
                for c, v in zip(cols, vals):
                    plsc.store_scatter(rows_v, [rows, c], v)
        pltpu.sync_copy(rows_v,
                        out_hbm.at[wid // 2, pl.ds((wid % 2) * _RPW, _RPW)])

    return _sc_gather


def kernel(obs, W1, b1, W2, b2, W3, b3, codebook):
    obs_flat = obs.reshape(_ROWS, _IN)
    idx3, loss_sum = _tc_main(
        obs_flat, W1, b1.reshape(1, _HID), W2, b2.reshape(1, _HID),
        W3, b3.reshape(1, _OUT), codebook)
    idx_flat = idx3.reshape(_ROWS)
    msg = _make_sc_gather()(codebook, idx_flat)
    idx_out = idx3.reshape(_B, _N)
    cmt_loss = loss_sum[0, 0] / jnp.float32(_ROWS * _OUT)
    return (msg, idx_out, cmt_loss)
